# baseline (speedup 1.0000x reference)
import dataclasses
import functools

import jax
import jax.numpy as jnp
from jax import lax
from jax.experimental import pallas as pl
from jax.experimental.pallas import tpu as pltpu
from jax.experimental.pallas import tpu_sc as plsc

_FOLD_CHUNK = 2000
_NIDX = 4


def _sc_compiler_params():
    cp = pltpu.CompilerParams(use_tc_tiling_on_sc=False)
    if "needs_layout_passes" in pltpu.CompilerParams.__dataclass_fields__:
        cp = dataclasses.replace(cp, needs_layout_passes=False)
    return cp


def _pair_halves(w0, w1, shift):
    if shift:
        w0 = lax.shift_right_logical(w0, shift)
        w1 = lax.shift_right_logical(w1, shift)
    packed = plsc.pack(w0, w1, format=plsc.PackFormat.INTERLEAVED)
    return plsc.bitcast(packed, jnp.bfloat16)


def _sc_fused(emb1, idx1, base_token, V, D, B, L, l0, l1):
    mesh = plsc.VectorSubcoreMesh(core_axis_name="core", subcore_axis_name="subcore")

    @functools.partial(
        pl.kernel,
        out_type=jax.ShapeDtypeStruct(((l1 - l0) * (D // 2) * B,), jnp.int32),
        mesh=mesh,
        compiler_params=_sc_compiler_params(),
        scratch_types=[
            pltpu.VMEM((V,), jnp.int32),
            pltpu.VMEM((_FOLD_CHUNK,), jnp.float32),
            pltpu.VMEM((_FOLD_CHUNK,), jnp.float32),
            pltpu.VMEM((_NIDX, B), jnp.int32),
            pltpu.VMEM((2, B), jnp.int32),
            pltpu.VMEM((D,), jnp.float32),
            pltpu.SemaphoreType.DMA,
            pltpu.SemaphoreType.DMA,
            pltpu.SemaphoreType.DMA,
            pltpu.SemaphoreType.DMA,
            pltpu.SemaphoreType.DMA,
            pltpu.SemaphoreType.DMA,
            pltpu.SemaphoreType.DMA,
        ],
    )
    def kern(emb_hbm, idx_hbm, base_hbm, out_hbm,
             tbl_v, lo_v, hi_v, idx_v, out_v, base_v,
             s_i0, s_i1, s_i2, s_i3, s_o0, s_o1, s_f):
        core = lax.axis_index("core")
        sub = lax.axis_index("subcore")
        t = sub * 2 + core
        r0 = 2 * t

        pltpu.make_async_copy(base_hbm, base_v, s_f).start()
        pltpu.make_async_copy(base_hbm, base_v, s_f).wait()
        vlo = plsc.load_gather(base_v, [lax.broadcast(r0, (16,))])
        vhi = plsc.load_gather(base_v, [lax.broadcast(r0 + 1, (16,))])
        base_pair = plsc.pack(vlo, vhi, format=plsc.PackFormat.INTERLEAVED)

        @pl.loop(0, V // _FOLD_CHUNK)
        def _(k):
            off = k * _FOLD_CHUNK
            pltpu.make_async_copy(
                emb_hbm.at[pl.ds(r0 * V + off, _FOLD_CHUNK)], lo_v, s_f).start()
            pltpu.make_async_copy(
                emb_hbm.at[pl.ds((r0 + 1) * V + off, _FOLD_CHUNK)], hi_v, s_f).start()
            pltpu.make_async_copy(
                emb_hbm.at[pl.ds(r0 * V + off, _FOLD_CHUNK)], lo_v, s_f).wait()
            pltpu.make_async_copy(
                emb_hbm.at[pl.ds((r0 + 1) * V + off, _FOLD_CHUNK)], hi_v, s_f).wait()

            @plsc.parallel_loop(0, _FOLD_CHUNK // 16, unroll=4)
            def _(i):
                a = lo_v[pl.ds(i * 16, 16)]
                b = hi_v[pl.ds(i * 16, 16)]
                pv = plsc.pack(a, b, format=plsc.PackFormat.INTERLEAVED) + base_pair
                tbl_v[pl.ds(off + i * 16, 16)] = plsc.bitcast(pv, jnp.int32)

        idx_sems = (s_i0, s_i1, s_i2, s_i3)
        out_sems = (s_o0, s_o1)

        def do_row(rel, j):
            l = l0 + rel
            pre = (j + _NIDX - 1) % _NIDX

            @pl.when(rel + _NIDX - 1 < l1 - l0)
            def _():
                pltpu.make_async_copy(
                    idx_hbm.at[pl.ds((l + _NIDX - 1) * B, B)], idx_v.at[pre],
                    idx_sems[pre]).start()

            pltpu.make_async_copy(
                idx_hbm.at[pl.ds(l * B, B)], idx_v.at[j], idx_sems[j]).wait()

            ob = j % 2

            @pl.when(rel >= 2)
            def _():
                pltpu.make_async_copy(
                    out_v.at[ob],
                    out_hbm.at[pl.ds(((rel - 2) * (D // 2) + t) * B, B)],
                    out_sems[ob]).wait()

            @plsc.parallel_loop(0, B // 128)
            def _(bt):
                for q in range(8):
                    goff = bt * 128 + q * 16
                    iv = idx_v[j, pl.ds(goff, 16)]
                    out_v[ob, pl.ds(goff, 16)] = plsc.load_gather(tbl_v, [iv])

            pltpu.make_async_copy(
                out_v.at[ob], out_hbm.at[pl.ds((rel * (D // 2) + t) * B, B)],
                out_sems[ob]).start()

        for j in range(_NIDX - 1):
            pltpu.make_async_copy(
                idx_hbm.at[pl.ds((l0 + j) * B, B)], idx_v.at[j],
                idx_sems[j]).start()

        @pl.loop(0, l1 - l0, step=_NIDX)
        def _(rel):
            for j in range(_NIDX):
                do_row(rel + j, j)

        HL = l1 - l0
        pltpu.make_async_copy(
            out_v.at[0], out_hbm.at[pl.ds(((HL - 2) * (D // 2) + t) * B, B)],
            out_sems[0]).wait()
        pltpu.make_async_copy(
            out_v.at[1], out_hbm.at[pl.ds(((HL - 1) * (D // 2) + t) * B, B)],
            out_sems[1]).wait()

    return kern(emb1, idx1, base_token)


def _tc_unpack_first(words, D, B, L, HL):
    def body(x_ref, o_ref):
        x = x_ref[...].reshape(D // 2, B)
        o_ref[...] = pltpu.bitcast(x, jnp.bfloat16).reshape(1, D, B)

    return pl.pallas_call(
        body,
        grid=(HL,),
        in_specs=[pl.BlockSpec((D // 2 * B,), lambda l: (l,))],
        out_specs=pl.BlockSpec((1, D, B), lambda l: (l, 0, 0)),
        out_shape=jax.ShapeDtypeStruct((L, D, B), jnp.bfloat16),
    )(words)


def _tc_unpack_second(acc, words, D, B, L, HL):
    def body(acc_ref, x_ref, o_ref):
        x = x_ref[...].reshape(D // 2, B)
        o_ref[...] = pltpu.bitcast(x, jnp.bfloat16).reshape(1, D, B)

    return pl.pallas_call(
        body,
        grid=(L - HL,),
        in_specs=[
            pl.BlockSpec((1, 8, 128), lambda l: (0, 0, 0)),
            pl.BlockSpec((D // 2 * B,), lambda l: (l,)),
        ],
        out_specs=pl.BlockSpec((1, D, B), lambda l: (l + HL, 0, 0)),
        out_shape=jax.ShapeDtypeStruct((L, D, B), jnp.bfloat16),
        input_output_aliases={0: 0},
    )(acc, words)


def kernel(actions, batch_time_shape, embedding, base_token):
    V, D = embedding.shape
    B, L = actions.shape
    HL = L // 2

    emb1 = embedding.T.reshape(V * D)
    idx1 = actions.T.reshape(B * L)
    words1 = _sc_fused(emb1, idx1, base_token, V, D, B, L, 0, HL)
    words2 = _sc_fused(emb1, idx1, base_token, V, D, B, L, HL, L)
    acc = _tc_unpack_first(words1, D, B, L, HL)
    out3 = _tc_unpack_second(acc, words2, D, B, L, HL)
    return out3.transpose(2, 0, 1)


# speedup vs baseline: 1.0126x; 1.0126x over previous
import dataclasses
import functools

import jax
import jax.numpy as jnp
from jax import lax
from jax.experimental import pallas as pl
from jax.experimental.pallas import tpu as pltpu
from jax.experimental.pallas import tpu_sc as plsc

_FOLD_CHUNK = 2000
_NIDX = 4


def _sc_compiler_params():
    cp = pltpu.CompilerParams(use_tc_tiling_on_sc=False)
    if "needs_layout_passes" in pltpu.CompilerParams.__dataclass_fields__:
        cp = dataclasses.replace(cp, needs_layout_passes=False)
    return cp


def _pair_halves(w0, w1, shift):
    if shift:
        w0 = lax.shift_right_logical(w0, shift)
        w1 = lax.shift_right_logical(w1, shift)
    packed = plsc.pack(w0, w1, format=plsc.PackFormat.INTERLEAVED)
    return plsc.bitcast(packed, jnp.bfloat16)


def _sc_fused(emb1, idx1, base_token, V, D, B, L):
    mesh = plsc.VectorSubcoreMesh(core_axis_name="core", subcore_axis_name="subcore")

    @functools.partial(
        pl.kernel,
        out_type=jax.ShapeDtypeStruct((L * (D // 2) * B,), jnp.int32),
        mesh=mesh,
        compiler_params=_sc_compiler_params(),
        scratch_types=[
            pltpu.VMEM((V,), jnp.int32),
            pltpu.VMEM((_FOLD_CHUNK,), jnp.float32),
            pltpu.VMEM((_FOLD_CHUNK,), jnp.float32),
            pltpu.VMEM((_NIDX, B), jnp.int32),
            pltpu.VMEM((2, B), jnp.int32),
            pltpu.VMEM((D,), jnp.float32),
            pltpu.SemaphoreType.DMA,
            pltpu.SemaphoreType.DMA,
            pltpu.SemaphoreType.DMA,
            pltpu.SemaphoreType.DMA,
            pltpu.SemaphoreType.DMA,
            pltpu.SemaphoreType.DMA,
            pltpu.SemaphoreType.DMA,
        ],
    )
    def kern(emb_hbm, idx_hbm, base_hbm, out_hbm,
             tbl_v, lo_v, hi_v, idx_v, out_v, base_v,
             s_i0, s_i1, s_i2, s_i3, s_o0, s_o1, s_f):
        core = lax.axis_index("core")
        sub = lax.axis_index("subcore")
        t = sub * 2 + core
        r0 = 2 * t

        pltpu.make_async_copy(base_hbm, base_v, s_f).start()
        pltpu.make_async_copy(base_hbm, base_v, s_f).wait()
        vlo = plsc.load_gather(base_v, [lax.broadcast(r0, (16,))])
        vhi = plsc.load_gather(base_v, [lax.broadcast(r0 + 1, (16,))])
        base_pair = plsc.pack(vlo, vhi, format=plsc.PackFormat.INTERLEAVED)

        @pl.loop(0, V // _FOLD_CHUNK)
        def _(k):
            off = k * _FOLD_CHUNK
            pltpu.make_async_copy(
                emb_hbm.at[pl.ds(r0 * V + off, _FOLD_CHUNK)], lo_v, s_f).start()
            pltpu.make_async_copy(
                emb_hbm.at[pl.ds((r0 + 1) * V + off, _FOLD_CHUNK)], hi_v, s_f).start()
            pltpu.make_async_copy(
                emb_hbm.at[pl.ds(r0 * V + off, _FOLD_CHUNK)], lo_v, s_f).wait()
            pltpu.make_async_copy(
                emb_hbm.at[pl.ds((r0 + 1) * V + off, _FOLD_CHUNK)], hi_v, s_f).wait()

            @plsc.parallel_loop(0, _FOLD_CHUNK // 16, unroll=4)
            def _(i):
                a = lo_v[pl.ds(i * 16, 16)]
                b = hi_v[pl.ds(i * 16, 16)]
                pv = plsc.pack(a, b, format=plsc.PackFormat.INTERLEAVED) + base_pair
                tbl_v[pl.ds(off + i * 16, 16)] = plsc.bitcast(pv, jnp.int32)

        idx_sems = (s_i0, s_i1, s_i2, s_i3)
        out_sems = (s_o0, s_o1)

        def do_row(l, j):
            pre = (j + _NIDX - 1) % _NIDX

            @pl.when(l + _NIDX - 1 < L)
            def _():
                pltpu.make_async_copy(
                    idx_hbm.at[pl.ds((l + _NIDX - 1) * B, B)], idx_v.at[pre],
                    idx_sems[pre]).start()

            pltpu.make_async_copy(
                idx_hbm.at[pl.ds(l * B, B)], idx_v.at[j], idx_sems[j]).wait()

            ob = j % 2

            @pl.when(l >= 2)
            def _():
                pltpu.make_async_copy(
                    out_v.at[ob],
                    out_hbm.at[pl.ds(((l - 2) * (D // 2) + t) * B, B)],
                    out_sems[ob]).wait()

            @plsc.parallel_loop(0, B // 128)
            def _(bt):
                for q in range(8):
                    goff = bt * 128 + q * 16
                    iv = idx_v[j, pl.ds(goff, 16)]
                    out_v[ob, pl.ds(goff, 16)] = plsc.load_gather(tbl_v, [iv])

            pltpu.make_async_copy(
                out_v.at[ob], out_hbm.at[pl.ds((l * (D // 2) + t) * B, B)],
                out_sems[ob]).start()

        for j in range(_NIDX - 1):
            pltpu.make_async_copy(
                idx_hbm.at[pl.ds(j * B, B)], idx_v.at[j], idx_sems[j]).start()

        @pl.loop(0, L, step=_NIDX)
        def _(l):
            for j in range(_NIDX):
                do_row(l + j, j)

        pltpu.make_async_copy(
            out_v.at[0], out_hbm.at[pl.ds(((L - 2) * (D // 2) + t) * B, B)],
            out_sems[0]).wait()
        pltpu.make_async_copy(
            out_v.at[1], out_hbm.at[pl.ds(((L - 1) * (D // 2) + t) * B, B)],
            out_sems[1]).wait()

    return kern(emb1, idx1, base_token)


def _tc_unpack(words, D, B, L):
    def body(x_ref, o_ref):
        x = x_ref[...].reshape(D // 2, B)
        o_ref[...] = pltpu.bitcast(x, jnp.bfloat16).reshape(1, D, B)

    return pl.pallas_call(
        body,
        grid=(L,),
        in_specs=[pl.BlockSpec((D // 2 * B,), lambda l: (l,))],
        out_specs=pl.BlockSpec((1, D, B), lambda l: (l, 0, 0)),
        out_shape=jax.ShapeDtypeStruct((L, D, B), jnp.bfloat16),
    )(words)


def kernel(actions, batch_time_shape, embedding, base_token):
    V, D = embedding.shape
    B, L = actions.shape

    emb1 = embedding.T.reshape(V * D)
    idx1 = actions.T.reshape(B * L)
    words = _sc_fused(emb1, idx1, base_token, V, D, B, L)
    out3 = _tc_unpack(words, D, B, L)
    return out3.transpose(2, 0, 1)
